# baseline (speedup 1.0000x reference)
import functools

import jax
import jax.numpy as jnp
from jax import lax
from jax.experimental import pallas as pl
from jax.experimental.pallas import tpu as pltpu
from jax.experimental.pallas import tpu_sc as plsc

N = 10000
E = 320000
D = 128

NC = 2
NS = 16
NW = NC * NS
EPW = E // NW
CH = 80
NCHUNK = EPW // CH

ACC_PAD = 10240
RPT = ACC_PAD // NS
ZR = 40
CNT_PAD = 10240
CPT = CNT_PAD // NS
ZC = 160


def _make_sc_pass(with_count):
  mesh = plsc.VectorSubcoreMesh(
      core_axis_name="c", subcore_axis_name="s", num_cores=NC, num_subcores=NS)

  acc_type = jax.ShapeDtypeStruct((ACC_PAD, D), jnp.float32)
  cnt_type = jax.ShapeDtypeStruct((CNT_PAD,), jnp.float32)
  if with_count:
    out_type = [acc_type, acc_type, cnt_type, cnt_type]
  else:
    out_type = [acc_type, acc_type]

  scratch = [
      pltpu.VMEM_SHARED((ACC_PAD, D), jnp.float32),
      pltpu.VMEM((ZR, D), jnp.float32),
      pltpu.VMEM((EPW,), jnp.int32),
      pltpu.VMEM((CH,), jnp.int32),
      pltpu.VMEM((CH,), jnp.int32),
      pltpu.VMEM((CH, D), jnp.float32),
      pltpu.VMEM((CH, D), jnp.float32),
      pltpu.SemaphoreType.DMA,
      pltpu.SemaphoreType.DMA,
      pltpu.SemaphoreType.DMA,
      pltpu.SemaphoreType.DMA,
      pltpu.SemaphoreType.DMA,
  ]
  if with_count:
    scratch += [
        pltpu.VMEM_SHARED((CNT_PAD,), jnp.float32),
        pltpu.VMEM((ZC,), jnp.float32),
        pltpu.VMEM((CH,), jnp.float32),
        pltpu.SemaphoreType.DMA,
        pltpu.SemaphoreType.DMA,
    ]

  def body(x_hbm, ei_hbm, *rest):
    if with_count:
      (acc0_out, acc1_out, cnt0_out, cnt1_out, acc_sp, zbuf, src_all,
       dst_v0, dst_v1, rows_v0, rows_v1, semg0, semg1, semd0, semd1, semz,
       cnt_sp, zcnt, ones_v, semc0, semc1) = rest
    else:
      (acc0_out, acc1_out, acc_sp, zbuf, src_all, dst_v0, dst_v1,
       rows_v0, rows_v1, semg0, semg1, semd0, semd1, semz) = rest
      cnt_sp = zcnt = ones_v = semc0 = semc1 = None
    dst_v = [dst_v0, dst_v1]
    rows_v = [rows_v0, rows_v1]
    semg = [semg0, semg1]
    semd = [semd0, semd1]
    semc = [semc0, semc1]

    c = lax.axis_index("c")
    s = lax.axis_index("s")
    wid = s * NC + c
    ebase = wid * EPW

    zeros16 = jnp.zeros((16,), jnp.float32)
    for r in range(ZR):
      for j in range(D // 16):
        zbuf[r, pl.ds(j * 16, 16)] = zeros16

    def zero_acc(i, _):
      pltpu.async_copy(zbuf, acc_sp.at[pl.ds(s * RPT + i * ZR, ZR)], semz)
      return 0
    lax.fori_loop(0, RPT // ZR, zero_acc, 0)

    if with_count:
      for j in range(ZC // 16):
        zcnt[pl.ds(j * 16, 16)] = zeros16
      for j in range(CH // 16):
        ones_v[pl.ds(j * 16, 16)] = zeros16 + 1.0

      def zero_cnt(i, _):
        pltpu.async_copy(zcnt, cnt_sp.at[pl.ds(s * CPT + i * ZC, ZC)], semz)
        return 0
      lax.fori_loop(0, CPT // ZC, zero_cnt, 0)

    pltpu.async_copy(ei_hbm.at[pl.ds(ebase, EPW)], src_all, semz)
    pltpu.make_async_copy(ei_hbm.at[pl.ds(ebase, EPW)], src_all, semz).wait()

    def drain_zero(i, _):
      pltpu.make_async_copy(zbuf, acc_sp.at[pl.ds(s * RPT + i * ZR, ZR)],
                            semz).wait()
      return 0
    lax.fori_loop(0, RPT // ZR, drain_zero, 0)
    if with_count:
      def drain_zcnt(i, _):
        pltpu.make_async_copy(zcnt, cnt_sp.at[pl.ds(s * CPT + i * ZC, ZC)],
                              semz).wait()
        return 0
      lax.fori_loop(0, CPT // ZC, drain_zcnt, 0)

    plsc.subcore_barrier()

    def start_gather(i, p):
      pltpu.async_copy(x_hbm.at[src_all.at[pl.ds(i * CH, CH)]], rows_v[p],
                       semg[p])

    def wait_gather(i, p):
      pltpu.make_async_copy(x_hbm.at[src_all.at[pl.ds(i * CH, CH)]],
                            rows_v[p], semg[p]).wait()

    def start_dst(i, p):
      pltpu.async_copy(ei_hbm.at[pl.ds(E + ebase + i * CH, CH)], dst_v[p],
                       semd[p])

    def wait_dst(i, p):
      pltpu.make_async_copy(ei_hbm.at[pl.ds(E + ebase + i * CH, CH)],
                            dst_v[p], semd[p]).wait()

    def start_cnt(p):
      pltpu.async_copy(ones_v, cnt_sp.at[dst_v[p]], semc[p], add=True)

    def wait_cnt(p):
      pltpu.make_async_copy(ones_v, cnt_sp.at[dst_v[p]], semc[p]).wait()

    def step(i, p, first=False, last=False):
      wait_gather(i, p)
      wait_dst(i, p)
      pltpu.sync_copy(rows_v[p], acc_sp.at[dst_v[p]], add=True)
      if with_count:
        start_cnt(p)
      if not last:
        start_gather(i + 2, p)
      if with_count and not first:
        wait_cnt(1 - p)
      if not last:
        start_dst(i + 1, 1 - p)

    start_dst(0, 0)
    start_gather(0, 0)
    start_gather(1, 1)
    step(0, 0, first=True)

    def chunk_pair(k, _):
      step(2 * k + 1, 1)
      step(2 * k + 2, 0)
      return 0
    lax.fori_loop(0, (NCHUNK - 3) // 2, chunk_pair, 0)

    i = NCHUNK - 2
    wait_gather(i, 1)
    wait_dst(i, 1)
    pltpu.sync_copy(rows_v[1], acc_sp.at[dst_v[1]], add=True)
    if with_count:
      start_cnt(1)
      wait_cnt(0)
    start_dst(i + 1, 0)
    i = NCHUNK - 1
    wait_gather(i, 0)
    wait_dst(i, 0)
    pltpu.sync_copy(rows_v[0], acc_sp.at[dst_v[0]], add=True)
    if with_count:
      start_cnt(0)
      wait_cnt(1)
      wait_cnt(0)

    plsc.subcore_barrier()

    @pl.when(c == 0)
    def _():
      pltpu.sync_copy(acc_sp.at[pl.ds(s * RPT, RPT)],
                      acc0_out.at[pl.ds(s * RPT, RPT)])
    @pl.when(c == 1)
    def _():
      pltpu.sync_copy(acc_sp.at[pl.ds(s * RPT, RPT)],
                      acc1_out.at[pl.ds(s * RPT, RPT)])
    if with_count:
      @pl.when(c == 0)
      def _():
        pltpu.sync_copy(cnt_sp.at[pl.ds(s * CPT, CPT)],
                        cnt0_out.at[pl.ds(s * CPT, CPT)])
      @pl.when(c == 1)
      def _():
        pltpu.sync_copy(cnt_sp.at[pl.ds(s * CPT, CPT)],
                        cnt1_out.at[pl.ds(s * CPT, CPT)])

  return pl.kernel(body, out_type=out_type, mesh=mesh, scratch_types=scratch)


_sc_pass_cnt = _make_sc_pass(True)
_sc_pass = _make_sc_pass(False)


BR = 1000


def _mm_body(relu, a0, a1, x, c0, c1, wrel, wroot, b, o):
  cnt = jnp.maximum(c0[...] + c1[...], 1.0)
  agg = (a0[...] + a1[...]) / cnt
  y = (jnp.dot(agg, wrel[...], preferred_element_type=jnp.float32)
       + jnp.dot(x[...], wroot[...], preferred_element_type=jnp.float32)
       + b[...])
  o[...] = jnp.maximum(y, 0.0) if relu else y


def _make_mm(relu):
  row_spec = pl.BlockSpec((BR, D), lambda i: (i, 0))
  acc_spec = pl.BlockSpec((BR, D), lambda i: (i, 0))
  cnt_spec = pl.BlockSpec((BR, 1), lambda i: (i, 0))
  w_spec = pl.BlockSpec((D, D), lambda i: (0, 0))
  b_spec = pl.BlockSpec((1, D), lambda i: (0, 0))
  return pl.pallas_call(
      functools.partial(_mm_body, relu),
      grid=(N // BR,),
      in_specs=[acc_spec, acc_spec, row_spec, cnt_spec, cnt_spec,
                w_spec, w_spec, b_spec],
      out_specs=row_spec,
      out_shape=jax.ShapeDtypeStruct((N, D), jnp.float32),
  )


_mm_relu = _make_mm(True)
_mm_plain = _make_mm(False)


@jax.jit
def kernel(x, edge_index, W1_rel, W1_root, b1, W2_rel, W2_root, b2):
  ei = edge_index.reshape(2 * E)
  a0, a1, cnt0, cnt1 = _sc_pass_cnt(x, ei)
  c0 = cnt0[:, None]
  c1 = cnt1[:, None]
  h = _mm_relu(a0, a1, x, c0, c1, W1_rel, W1_root, b1[None, :])
  b0, b1_ = _sc_pass(h, ei)
  out = _mm_plain(b0, b1_, h, c0, c1, W2_rel, W2_root, b2[None, :])
  return out


# speedup vs baseline: 1.0584x; 1.0584x over previous
import functools

import jax
import jax.numpy as jnp
from jax import lax
from jax.experimental import pallas as pl
from jax.experimental.pallas import tpu as pltpu
from jax.experimental.pallas import tpu_sc as plsc

N = 10000
E = 320000
D = 128

NC = 2
NS = 16
NW = NC * NS
EPW = E // NW
CH = 80
NCHUNK = EPW // CH

ACC_PAD = 10240
RPT = ACC_PAD // NS
ZR = 40
CNT_PAD = 10240
CPT = CNT_PAD // NS
ZC = 160


def _make_sc_pass(with_count):
  mesh = plsc.VectorSubcoreMesh(
      core_axis_name="c", subcore_axis_name="s", num_cores=NC, num_subcores=NS)

  acc_type = jax.ShapeDtypeStruct((ACC_PAD, D), jnp.float32)
  cnt_type = jax.ShapeDtypeStruct((CNT_PAD,), jnp.float32)
  if with_count:
    out_type = [acc_type, acc_type, cnt_type, cnt_type]
  else:
    out_type = [acc_type, acc_type]

  scratch = [
      pltpu.VMEM_SHARED((ACC_PAD, D), jnp.float32),
      pltpu.VMEM((ZR, D), jnp.float32),
      pltpu.VMEM((EPW,), jnp.int32),
      pltpu.VMEM((CH,), jnp.int32),
      pltpu.VMEM((CH,), jnp.int32),
      pltpu.VMEM((CH, D), jnp.float32),
      pltpu.VMEM((CH, D), jnp.float32),
      pltpu.SemaphoreType.DMA,
      pltpu.SemaphoreType.DMA,
      pltpu.SemaphoreType.DMA,
      pltpu.SemaphoreType.DMA,
      pltpu.SemaphoreType.DMA,
  ]
  if with_count:
    scratch += [
        pltpu.VMEM_SHARED((CNT_PAD,), jnp.float32),
        pltpu.VMEM((ZC,), jnp.float32),
        pltpu.VMEM((CH,), jnp.float32),
        pltpu.SemaphoreType.DMA,
        pltpu.SemaphoreType.DMA,
    ]

  def body(x_hbm, ei_hbm, *rest):
    if with_count:
      (acc0_out, acc1_out, cnt0_out, cnt1_out, acc_sp, zbuf, src_all,
       dst_v0, dst_v1, rows_v0, rows_v1, semg0, semg1, semd0, semd1, semz,
       cnt_sp, zcnt, ones_v, semc0, semc1) = rest
    else:
      (acc0_out, acc1_out, acc_sp, zbuf, src_all, dst_v0, dst_v1,
       rows_v0, rows_v1, semg0, semg1, semd0, semd1, semz) = rest
      cnt_sp = zcnt = ones_v = semc0 = semc1 = None
    dst_v = [dst_v0, dst_v1]
    rows_v = [rows_v0, rows_v1]
    semg = [semg0, semg1]
    semd = [semd0, semd1]
    semc = [semc0, semc1]

    c = lax.axis_index("c")
    s = lax.axis_index("s")
    wid = s * NC + c
    ebase = wid * EPW

    zeros16 = jnp.zeros((16,), jnp.float32)
    for r in range(ZR):
      for j in range(D // 16):
        zbuf[r, pl.ds(j * 16, 16)] = zeros16

    def zero_acc(i, _):
      pltpu.async_copy(zbuf, acc_sp.at[pl.ds(s * RPT + i * ZR, ZR)], semz)
      return 0
    lax.fori_loop(0, RPT // ZR, zero_acc, 0)

    if with_count:
      for j in range(ZC // 16):
        zcnt[pl.ds(j * 16, 16)] = zeros16
      for j in range(CH // 16):
        ones_v[pl.ds(j * 16, 16)] = zeros16 + 1.0

      def zero_cnt(i, _):
        pltpu.async_copy(zcnt, cnt_sp.at[pl.ds(s * CPT + i * ZC, ZC)], semz)
        return 0
      lax.fori_loop(0, CPT // ZC, zero_cnt, 0)

    pltpu.sync_copy(ei_hbm.at[pl.ds(ebase, EPW)], src_all)

    def drain_zero(i, _):
      pltpu.make_async_copy(zbuf, acc_sp.at[pl.ds(s * RPT + i * ZR, ZR)],
                            semz).wait()
      return 0
    lax.fori_loop(0, RPT // ZR, drain_zero, 0)
    if with_count:
      def drain_zcnt(i, _):
        pltpu.make_async_copy(zcnt, cnt_sp.at[pl.ds(s * CPT + i * ZC, ZC)],
                              semz).wait()
        return 0
      lax.fori_loop(0, CPT // ZC, drain_zcnt, 0)

    plsc.subcore_barrier()

    def start_gather(i, p):
      pltpu.async_copy(x_hbm.at[src_all.at[pl.ds(i * CH, CH)]], rows_v[p],
                       semg[p])

    def wait_gather(i, p):
      pltpu.make_async_copy(x_hbm.at[src_all.at[pl.ds(i * CH, CH)]],
                            rows_v[p], semg[p]).wait()

    def start_dst(i, p):
      pltpu.async_copy(ei_hbm.at[pl.ds(E + ebase + i * CH, CH)], dst_v[p],
                       semd[p])

    def wait_dst(i, p):
      pltpu.make_async_copy(ei_hbm.at[pl.ds(E + ebase + i * CH, CH)],
                            dst_v[p], semd[p]).wait()

    def start_cnt(p):
      pltpu.async_copy(ones_v, cnt_sp.at[dst_v[p]], semc[p], add=True)

    def wait_cnt(p):
      pltpu.make_async_copy(ones_v, cnt_sp.at[dst_v[p]], semc[p]).wait()

    def step(i, p, first=False, last=False):
      wait_gather(i, p)
      wait_dst(i, p)
      pltpu.sync_copy(rows_v[p], acc_sp.at[dst_v[p]], add=True)
      if with_count:
        start_cnt(p)
      if not last:
        start_gather(i + 2, p)
      if with_count and not first:
        wait_cnt(1 - p)
      if not last:
        start_dst(i + 1, 1 - p)

    start_dst(0, 0)
    start_gather(0, 0)
    start_gather(1, 1)
    step(0, 0, first=True)

    def chunk_pair(k, _):
      step(2 * k + 1, 1)
      step(2 * k + 2, 0)
      return 0
    lax.fori_loop(0, (NCHUNK - 3) // 2, chunk_pair, 0)

    i = NCHUNK - 2
    wait_gather(i, 1)
    wait_dst(i, 1)
    pltpu.sync_copy(rows_v[1], acc_sp.at[dst_v[1]], add=True)
    if with_count:
      start_cnt(1)
      wait_cnt(0)
    start_dst(i + 1, 0)
    i = NCHUNK - 1
    wait_gather(i, 0)
    wait_dst(i, 0)
    pltpu.sync_copy(rows_v[0], acc_sp.at[dst_v[0]], add=True)
    if with_count:
      start_cnt(0)
      wait_cnt(1)
      wait_cnt(0)

    plsc.subcore_barrier()

    @pl.when(c == 0)
    def _():
      pltpu.sync_copy(acc_sp.at[pl.ds(s * RPT, RPT)],
                      acc0_out.at[pl.ds(s * RPT, RPT)])
    @pl.when(c == 1)
    def _():
      pltpu.sync_copy(acc_sp.at[pl.ds(s * RPT, RPT)],
                      acc1_out.at[pl.ds(s * RPT, RPT)])
    if with_count:
      @pl.when(c == 0)
      def _():
        pltpu.sync_copy(cnt_sp.at[pl.ds(s * CPT, CPT)],
                        cnt0_out.at[pl.ds(s * CPT, CPT)])
      @pl.when(c == 1)
      def _():
        pltpu.sync_copy(cnt_sp.at[pl.ds(s * CPT, CPT)],
                        cnt1_out.at[pl.ds(s * CPT, CPT)])

  return pl.kernel(body, out_type=out_type, mesh=mesh, scratch_types=scratch)


_sc_pass_cnt = _make_sc_pass(True)
_sc_pass = _make_sc_pass(False)


BR = 1000


def _mm_body(relu, a0, a1, x, c0, c1, wrel, wroot, b, o):
  cnt = jnp.maximum(c0[...] + c1[...], 1.0)
  agg = (a0[...] + a1[...]) / cnt
  y = (jnp.dot(agg, wrel[...], preferred_element_type=jnp.float32)
       + jnp.dot(x[...], wroot[...], preferred_element_type=jnp.float32)
       + b[...])
  o[...] = jnp.maximum(y, 0.0) if relu else y


def _make_mm(relu):
  row_spec = pl.BlockSpec((BR, D), lambda i: (i, 0))
  acc_spec = pl.BlockSpec((BR, D), lambda i: (i, 0))
  cnt_spec = pl.BlockSpec((BR, 1), lambda i: (i, 0))
  w_spec = pl.BlockSpec((D, D), lambda i: (0, 0))
  b_spec = pl.BlockSpec((1, D), lambda i: (0, 0))
  return pl.pallas_call(
      functools.partial(_mm_body, relu),
      grid=(N // BR,),
      in_specs=[acc_spec, acc_spec, row_spec, cnt_spec, cnt_spec,
                w_spec, w_spec, b_spec],
      out_specs=row_spec,
      out_shape=jax.ShapeDtypeStruct((N, D), jnp.float32),
  )


_mm_relu = _make_mm(True)
_mm_plain = _make_mm(False)


@jax.jit
def kernel(x, edge_index, W1_rel, W1_root, b1, W2_rel, W2_root, b2):
  ei = edge_index.reshape(2 * E)
  a0, a1, cnt0, cnt1 = _sc_pass_cnt(x, ei)
  c0 = cnt0[:, None]
  c1 = cnt1[:, None]
  h = _mm_relu(a0, a1, x, c0, c1, W1_rel, W1_root, b1[None, :])
  b0, b1_ = _sc_pass(h, ei)
  out = _mm_plain(b0, b1_, h, c0, c1, W2_rel, W2_root, b2[None, :])
  return out


# speedup vs baseline: 1.3031x; 1.2312x over previous
import functools

import jax
import jax.numpy as jnp
from jax import lax
from jax.experimental import pallas as pl
from jax.experimental.pallas import tpu as pltpu
from jax.experimental.pallas import tpu_sc as plsc

N = 10000
E = 320000
D = 128

NC = 2
NS = 16
NW = NC * NS
EPW = E // NW
CH = 80
NCHUNK = EPW // CH

ACC_PAD = 10240
RPT = ACC_PAD // NS
ZR = 40
CNT_PAD = 10240
CPT = CNT_PAD // NS
ZC = 160


def _make_sc_pass(with_count):
  mesh = plsc.VectorSubcoreMesh(
      core_axis_name="c", subcore_axis_name="s", num_cores=NC, num_subcores=NS)

  acc_type = jax.ShapeDtypeStruct((ACC_PAD, D), jnp.float32)
  cnt_type = jax.ShapeDtypeStruct((CNT_PAD,), jnp.float32)
  if with_count:
    out_type = [acc_type, acc_type, cnt_type, cnt_type]
  else:
    out_type = [acc_type, acc_type]

  scratch = [
      pltpu.VMEM_SHARED((ACC_PAD, D), jnp.float32),
      pltpu.VMEM((ZR, D), jnp.float32),
      pltpu.VMEM((EPW,), jnp.int32),
      pltpu.VMEM((CH,), jnp.int32),
      pltpu.VMEM((CH,), jnp.int32),
      pltpu.VMEM((CH,), jnp.int32),
      pltpu.VMEM((CH, D), jnp.float32),
      pltpu.VMEM((CH, D), jnp.float32),
      pltpu.VMEM((CH, D), jnp.float32),
      pltpu.SemaphoreType.DMA,
      pltpu.SemaphoreType.DMA,
      pltpu.SemaphoreType.DMA,
      pltpu.SemaphoreType.DMA,
      pltpu.SemaphoreType.DMA,
      pltpu.SemaphoreType.DMA,
      pltpu.SemaphoreType.DMA,
      pltpu.SemaphoreType.DMA,
      pltpu.SemaphoreType.DMA,
  ]
  if with_count:
    scratch += [
        pltpu.VMEM_SHARED((CNT_PAD,), jnp.float32),
        pltpu.VMEM((ZC,), jnp.float32),
        pltpu.VMEM((CH,), jnp.float32),
        pltpu.SemaphoreType.DMA,
        pltpu.SemaphoreType.DMA,
    ]

  def body(x_hbm, ei_hbm, *rest):
    if with_count:
      (acc0_out, acc1_out, cnt0_out, cnt1_out, acc_sp, zbuf, src_all,
       dst_v0, dst_v1, dst_v2, rows_v0, rows_v1, rows_v2,
       semg0, semg1, semg2, semd0, semd1, semd2, sems0, sems1, semz,
       cnt_sp, zcnt, ones_v, semc0, semc1) = rest
    else:
      (acc0_out, acc1_out, acc_sp, zbuf, src_all, dst_v0, dst_v1, dst_v2,
       rows_v0, rows_v1, rows_v2, semg0, semg1, semg2, semd0, semd1, semd2,
       sems0, sems1, semz) = rest
      cnt_sp = zcnt = ones_v = semc0 = semc1 = None
    dst_v = [dst_v0, dst_v1, dst_v2]
    rows_v = [rows_v0, rows_v1, rows_v2]
    semg = [semg0, semg1, semg2]
    semd = [semd0, semd1, semd2]
    sems = [sems0, sems1]
    semc = [semc0, semc1]

    c = lax.axis_index("c")
    s = lax.axis_index("s")
    wid = s * NC + c
    ebase = wid * EPW

    zeros16 = jnp.zeros((16,), jnp.float32)
    for r in range(ZR):
      for j in range(D // 16):
        zbuf[r, pl.ds(j * 16, 16)] = zeros16

    def zero_acc(i, _):
      pltpu.async_copy(zbuf, acc_sp.at[pl.ds(s * RPT + i * ZR, ZR)], semz)
      return 0
    lax.fori_loop(0, RPT // ZR, zero_acc, 0)

    if with_count:
      for j in range(ZC // 16):
        zcnt[pl.ds(j * 16, 16)] = zeros16
      for j in range(CH // 16):
        ones_v[pl.ds(j * 16, 16)] = zeros16 + 1.0

      def zero_cnt(i, _):
        pltpu.async_copy(zcnt, cnt_sp.at[pl.ds(s * CPT + i * ZC, ZC)], semz)
        return 0
      lax.fori_loop(0, CPT // ZC, zero_cnt, 0)

    pltpu.sync_copy(ei_hbm.at[pl.ds(ebase, EPW)], src_all)

    def drain_zero(i, _):
      pltpu.make_async_copy(zbuf, acc_sp.at[pl.ds(s * RPT + i * ZR, ZR)],
                            semz).wait()
      return 0
    lax.fori_loop(0, RPT // ZR, drain_zero, 0)
    if with_count:
      def drain_zcnt(i, _):
        pltpu.make_async_copy(zcnt, cnt_sp.at[pl.ds(s * CPT + i * ZC, ZC)],
                              semz).wait()
        return 0
      lax.fori_loop(0, CPT // ZC, drain_zcnt, 0)

    plsc.subcore_barrier()

    def start_gather(i, q):
      pltpu.async_copy(x_hbm.at[src_all.at[pl.ds(i * CH, CH)]], rows_v[q],
                       semg[q])

    def wait_gather(i, q):
      pltpu.make_async_copy(x_hbm.at[src_all.at[pl.ds(i * CH, CH)]],
                            rows_v[q], semg[q]).wait()

    def start_dst(i, q):
      pltpu.async_copy(ei_hbm.at[pl.ds(E + ebase + i * CH, CH)], dst_v[q],
                       semd[q])

    def wait_dst(i, q):
      pltpu.make_async_copy(ei_hbm.at[pl.ds(E + ebase + i * CH, CH)],
                            dst_v[q], semd[q]).wait()

    def start_scat(q, p):
      pltpu.async_copy(rows_v[q], acc_sp.at[dst_v[q]], sems[p], add=True)
      if with_count:
        pltpu.async_copy(ones_v, cnt_sp.at[dst_v[q]], semc[p], add=True)

    def wait_scat(q, p):
      pltpu.make_async_copy(rows_v[q], acc_sp.at[dst_v[q]], sems[p]).wait()
      if with_count:
        pltpu.make_async_copy(ones_v, cnt_sp.at[dst_v[q]], semc[p]).wait()

    def step(i, q, p, first=False, last=False):
      wait_gather(i, q)
      wait_dst(i, q)
      start_scat(q, p)
      if not first:
        wait_scat((q + 2) % 3, 1 - p)
      if not last:
        start_gather(i + 2, (q + 2) % 3)
        start_dst(i + 2, (q + 2) % 3)

    start_dst(0, 0)
    start_dst(1, 1)
    start_gather(0, 0)
    start_gather(1, 1)
    step(0, 0, 0, first=True)
    step(1, 1, 1)

    def chunk_six(k, _):
      base = 2 + 6 * k
      for t in range(6):
        step(base + t, (2 + t) % 3, t % 2)
      return 0
    lax.fori_loop(0, (NCHUNK - 5) // 6, chunk_six, 0)

    step(NCHUNK - 3, 2, 0)
    step(NCHUNK - 2, 0, 1, last=True)
    step(NCHUNK - 1, 1, 0, last=True)
    wait_scat(1, 0)

    plsc.subcore_barrier()

    @pl.when(c == 0)
    def _():
      pltpu.sync_copy(acc_sp.at[pl.ds(s * RPT, RPT)],
                      acc0_out.at[pl.ds(s * RPT, RPT)])
    @pl.when(c == 1)
    def _():
      pltpu.sync_copy(acc_sp.at[pl.ds(s * RPT, RPT)],
                      acc1_out.at[pl.ds(s * RPT, RPT)])
    if with_count:
      @pl.when(c == 0)
      def _():
        pltpu.sync_copy(cnt_sp.at[pl.ds(s * CPT, CPT)],
                        cnt0_out.at[pl.ds(s * CPT, CPT)])
      @pl.when(c == 1)
      def _():
        pltpu.sync_copy(cnt_sp.at[pl.ds(s * CPT, CPT)],
                        cnt1_out.at[pl.ds(s * CPT, CPT)])

  return pl.kernel(body, out_type=out_type, mesh=mesh, scratch_types=scratch)


_sc_pass_cnt = _make_sc_pass(True)
_sc_pass = _make_sc_pass(False)


BR = 1000


def _mm_body(relu, a0, a1, x, c0, c1, wrel, wroot, b, o):
  cnt = jnp.maximum(c0[...] + c1[...], 1.0)
  agg = (a0[...] + a1[...]) / cnt
  y = (jnp.dot(agg, wrel[...], preferred_element_type=jnp.float32)
       + jnp.dot(x[...], wroot[...], preferred_element_type=jnp.float32)
       + b[...])
  o[...] = jnp.maximum(y, 0.0) if relu else y


def _make_mm(relu):
  row_spec = pl.BlockSpec((BR, D), lambda i: (i, 0))
  acc_spec = pl.BlockSpec((BR, D), lambda i: (i, 0))
  cnt_spec = pl.BlockSpec((BR, 1), lambda i: (i, 0))
  w_spec = pl.BlockSpec((D, D), lambda i: (0, 0))
  b_spec = pl.BlockSpec((1, D), lambda i: (0, 0))
  return pl.pallas_call(
      functools.partial(_mm_body, relu),
      grid=(N // BR,),
      in_specs=[acc_spec, acc_spec, row_spec, cnt_spec, cnt_spec,
                w_spec, w_spec, b_spec],
      out_specs=row_spec,
      out_shape=jax.ShapeDtypeStruct((N, D), jnp.float32),
  )


_mm_relu = _make_mm(True)
_mm_plain = _make_mm(False)


@jax.jit
def kernel(x, edge_index, W1_rel, W1_root, b1, W2_rel, W2_root, b2):
  ei = edge_index.reshape(2 * E)
  a0, a1, cnt0, cnt1 = _sc_pass_cnt(x, ei)
  c0 = cnt0[:, None]
  c1 = cnt1[:, None]
  h = _mm_relu(a0, a1, x, c0, c1, W1_rel, W1_root, b1[None, :])
  b0, b1_ = _sc_pass(h, ei)
  out = _mm_plain(b0, b1_, h, c0, c1, W2_rel, W2_root, b2[None, :])
  return out
